# baseline (speedup 1.0000x reference)
import functools

import jax
import jax.numpy as jnp
from jax import lax
from jax.experimental import pallas as pl
from jax.experimental.pallas import tpu as pltpu
from jax.experimental.pallas import tpu_sc as plsc

N_NODES = 10000
D = 128
E = 160000
H = 256
LANES = 16

NC, NS = 2, 16
NW = NC * NS

SPLITS = (0, 40960, 81920, 122880, 160000)
BG = 40

EPT = E // NS
BS = 80
NCH_S = EPT // BS
RPT = 640
RPD = 80
NKD = RPT // RPD
NPAD = NS * RPT

BE = 1280
NBLK = E // BE


@functools.cache
def _build_gather(gbase, gsz):
    mesh = plsc.VectorSubcoreMesh(core_axis_name="c", subcore_axis_name="s")
    epw = gsz // NW
    nch = epw // BG
    assert epw % BG == 0 and epw % 8 == 0

    @functools.partial(
        pl.kernel,
        mesh=mesh,
        out_type=[jax.ShapeDtypeStruct((gsz, D), jnp.float32)] * 7,
        scratch_types=(
            [pltpu.VMEM((epw,), jnp.int32)] * 8
            + [pltpu.VMEM((BG, D), jnp.float32)] * 16
            + [pltpu.SemaphoreType.DMA] * 4
        ),
        compiler_params=pltpu.CompilerParams(needs_layout_passes=False),
    )
    def gather_kernel(user_hbm, item_hbm, qt_hbm,
                      src_i, dst_i, q0_i, q1_i, uq0_i, uq1_i, iq0_i, iq1_i,
                      o_src, o_dst, o_qs, o_uq0, o_uq1, o_iq0, o_iq1,
                      v_src, v_dst, v_q0, v_q1, v_uq0, v_uq1, v_iq0, v_iq1,
                      *bufs_and_sems):
        rows0 = bufs_and_sems[0:8]
        rows1 = bufs_and_sems[8:16]
        g0, g1, w0, w1 = bufs_and_sems[16:20]
        wid = lax.axis_index("s") * NC + lax.axis_index("c")
        base = wid * epw
        ivs = (v_src, v_dst, v_q0, v_q1, v_uq0, v_uq1, v_iq0, v_iq1)
        ihs = (src_i, dst_i, q0_i, q1_i, uq0_i, uq1_i, iq0_i, iq1_i)
        for iv, ih in zip(ivs, ihs):
            pltpu.sync_copy(ih.at[pl.ds(gbase + base, epw)], iv)

        tables = (user_hbm, item_hbm) + (qt_hbm,) * 6
        ohbm = (o_src, o_dst, o_qs, o_uq0, o_uq1, o_iq0, o_iq1)

        def fire_g(j, rows, sem):
            off = j * BG
            for tbl, iv, rv in zip(tables, ivs, rows):
                pltpu.async_copy(tbl.at[iv.at[pl.ds(off, BG)]], rv, sem)

        def wait_g(rows, sem):
            for rv in rows:
                pltpu.make_async_copy(qt_hbm.at[pl.ds(0, BG)], rv, sem).wait()

        def wbufs(rows):
            return (rows[0], rows[1], rows[2], rows[4], rows[5],
                    rows[6], rows[7])

        def fire_w(j, rows, sem):
            ob = base + j * BG
            for ov, rv in zip(ohbm, wbufs(rows)):
                pltpu.async_copy(rv, ov.at[pl.ds(ob, BG)], sem)

        def wait_w(rows, sem):
            for ov, rv in zip(ohbm, wbufs(rows)):
                pltpu.make_async_copy(rv, ov.at[pl.ds(0, BG)], sem).wait()

        def qsum(rows):
            rq0, rq1 = rows[2], rows[3]

            def row(r, c2):
                for c in range(D // LANES):
                    s = pl.ds(c * LANES, LANES)
                    rq0[r, s] = rq0[r, s] + rq1[r, s]
                return c2
            lax.fori_loop(0, BG, row, 0)

        fire_g(0, rows0, g0)
        nbody = (nch - 1) // 2 if nch % 2 == 1 else (nch - 2) // 2

        def body(i, carry):
            a = 2 * i
            b = a + 1

            @pl.when(i > 0)
            def _():
                wait_w(rows1, w1)
            fire_g(b, rows1, g1)
            wait_g(rows0, g0)
            qsum(rows0)
            fire_w(a, rows0, w0)
            wait_w(rows0, w0)
            fire_g(a + 2, rows0, g0)
            wait_g(rows1, g1)
            qsum(rows1)
            fire_w(b, rows1, w1)
            return carry
        lax.fori_loop(0, nbody, body, 0)

        if nch % 2 == 1:
            wait_w(rows1, w1)
            wait_g(rows0, g0)
            qsum(rows0)
            fire_w(nch - 1, rows0, w0)
            wait_w(rows0, w0)
        else:
            wait_w(rows1, w1)
            fire_g(nch - 1, rows1, g1)
            wait_g(rows0, g0)
            qsum(rows0)
            fire_w(nch - 2, rows0, w0)
            wait_g(rows1, g1)
            qsum(rows1)
            fire_w(nch - 1, rows1, w1)
            wait_w(rows0, w0)
            wait_w(rows1, w1)

    return gather_kernel


def _dense_body(se_ref, de_ref, qs_ref, u0_ref, u1_ref, i0_ref, i1_ref,
                mk_ref, wk_ref, bk_ref, w1_ref, b1_ref, w2_ref, b2_ref,
                o1_ref, o2_ref):
    src = se_ref[...]
    dst = de_ref[...]
    qs = qs_ref[...]
    u0 = u0_ref[...]
    u1 = u1_ref[...]
    i0 = i0_ref[...]
    i1 = i1_ref[...]
    mk = mk_ref[...]

    f32 = jnp.float32
    kkey = (jnp.dot(src, wk_ref[0:D, :], preferred_element_type=f32)
            + jnp.dot(u0, wk_ref[D:2 * D, :], preferred_element_type=f32)
            + jnp.dot(u1, wk_ref[2 * D:3 * D, :], preferred_element_type=f32)
            + bk_ref[...])
    inv_sqrt_d = 1.0 / (D ** 0.5)
    s0 = jnp.sum(i0 * kkey, axis=1, keepdims=True) * inv_sqrt_d
    s1 = jnp.sum(i1 * kkey, axis=1, keepdims=True) * inv_sqrt_d
    m = jnp.maximum(s0, s1)
    e0 = jnp.exp(s0 - m)
    e1 = jnp.exp(s1 - m)
    den = e0 + e1
    a0 = e0 / den
    a1 = e1 / den
    iqe = a0 * i0 + a1 * i1
    uqs = u0 + u1

    h = (jnp.dot(src, w1_ref[0:D, :], preferred_element_type=f32)
         + jnp.dot(dst, w1_ref[D:2 * D, :], preferred_element_type=f32)
         + jnp.dot(uqs, w1_ref[2 * D:3 * D, :], preferred_element_type=f32)
         + jnp.dot(iqe, w1_ref[3 * D:4 * D, :], preferred_element_type=f32)
         + b1_ref[...])
    h = jnp.maximum(h, 0.0)
    rec = jnp.dot(h, w2_ref[...], preferred_element_type=f32) + b2_ref[...]
    qe = jnp.where(mk > 0.5, rec, qs)
    o1_ref[...] = src + qe
    o2_ref[...] = dst - qe


@functools.cache
def _build_dense(esz):
    return pl.pallas_call(
        _dense_body,
        grid=(esz // BE,),
        in_specs=[pl.BlockSpec((BE, D), lambda i: (i, 0))] * 7
        + [pl.BlockSpec((BE, 1), lambda i: (i, 0)),
           pl.BlockSpec((3 * D, D), lambda i: (0, 0)),
           pl.BlockSpec((1, D), lambda i: (0, 0)),
           pl.BlockSpec((4 * D, H), lambda i: (0, 0)),
           pl.BlockSpec((1, H), lambda i: (0, 0)),
           pl.BlockSpec((H, D), lambda i: (0, 0)),
           pl.BlockSpec((1, D), lambda i: (0, 0))],
        out_specs=[pl.BlockSpec((BE, D), lambda i: (i, 0))] * 2,
        out_shape=[jax.ShapeDtypeStruct((esz, D), jnp.float32)] * 2,
    )


@functools.cache
def _build_scatter():
    mesh = plsc.VectorSubcoreMesh(core_axis_name="c", subcore_axis_name="s")

    @functools.partial(
        pl.kernel,
        mesh=mesh,
        out_type=[jax.ShapeDtypeStruct((N_NODES, D), jnp.float32),
                  jax.ShapeDtypeStruct((N_NODES, D), jnp.float32)],
        scratch_types=[
            pltpu.VMEM((1, BS), jnp.int32),
            pltpu.VMEM((1, BS), jnp.int32),
            pltpu.VMEM((BS, D), jnp.float32),
            pltpu.VMEM((BS, D), jnp.float32),
            pltpu.VMEM((NPAD,), jnp.float32),
            pltpu.VMEM((RPT,), jnp.float32),
            pltpu.VMEM((RPT,), jnp.float32),
            pltpu.VMEM_SHARED((N_NODES, D), jnp.float32),
        ] + [pltpu.VMEM_SHARED((NPAD,), jnp.float32)] * NS
          + [pltpu.SemaphoreType.DMA] * 4,
        compiler_params=pltpu.CompilerParams(needs_layout_passes=False),
    )
    def scatter_kernel(mu0, mu1, mu2, mu3, mi0, mi1, mi2, mi3, dst_f, src_f,
                       user_out, item_out,
                       idx0_v, idx1_v, msg0_v, msg1_v, cnt_v,
                       tmp_v, ctot_v, acc_sh, *stage_and_sems):
        out_v = msg0_v
        mu = (mu0, mu1, mu2, mu3)
        mi = (mi0, mi1, mi2, mi3)
        stage = stage_and_sems[:NS]
        l0, l1, s0, s1 = stage_and_sems[NS:NS + 4]
        core = lax.axis_index("c")
        tid = lax.axis_index("s")
        zero16 = jnp.zeros((LANES,), jnp.float32)
        one16 = jnp.ones((LANES,), jnp.float32)

        def zrow(r, c2):
            for c in range(D // LANES):
                out_v[r, pl.ds(c * LANES, LANES)] = zero16
            return c2
        lax.fori_loop(0, RPD, zrow, 0)

        def zcnt(r, c2):
            cnt_v[pl.ds(r * LANES, LANES)] = zero16
            return c2
        lax.fori_loop(0, NPAD // LANES, zcnt, 0)

        rbase = tid * RPT
        for k in range(NKD):
            @pl.when(rbase + k * RPD < N_NODES)
            def _():
                pltpu.sync_copy(out_v, acc_sh.at[pl.ds(rbase + k * RPD, RPD)])
        plsc.subcore_barrier()

        def side(m_parts, idx_hbm, out_hbm):
            ebase = tid * EPT

            def fire_l(j, iv, mv, sem):
                off = ebase + j * BS
                pltpu.async_copy(idx_hbm.at[pl.ds(off, BS)], iv.at[0], sem)
                for s in range(4):
                    @pl.when((off >= SPLITS[s]) & (off < SPLITS[s + 1]))
                    def _():
                        pltpu.async_copy(
                            m_parts[s].at[pl.ds(off - SPLITS[s], BS)], mv,
                            sem)

            def wait_l(iv, mv, sem):
                pltpu.make_async_copy(idx_hbm.at[pl.ds(0, BS)], iv.at[0],
                                      sem).wait()
                pltpu.make_async_copy(m_parts[0].at[pl.ds(0, BS)], mv,
                                      sem).wait()

            def counts(iv):
                for v in range(BS // LANES):
                    vec = iv[0, pl.ds(v * LANES, LANES)]
                    plsc.addupdate_scatter(cnt_v, [vec], one16)

            def fire_s(iv, mv, sem):
                pltpu.async_copy(mv, acc_sh.at[iv.at[0]], sem, add=True)

            def wait_s(mv, sem):
                pltpu.make_async_copy(m_parts[0].at[pl.ds(0, BS)], mv,
                                      sem).wait()

            fire_l(0, idx0_v, msg0_v, l0)

            def chunk2(i, c2):
                a = 2 * i

                @pl.when(i > 0)
                def _():
                    wait_s(msg1_v, s1)
                fire_l(a + 1, idx1_v, msg1_v, l1)
                wait_l(idx0_v, msg0_v, l0)
                counts(idx0_v)
                fire_s(idx0_v, msg0_v, s0)
                wait_s(msg0_v, s0)
                fire_l(a + 2, idx0_v, msg0_v, l0)
                wait_l(idx1_v, msg1_v, l1)
                counts(idx1_v)
                fire_s(idx1_v, msg1_v, s1)
                return c2
            lax.fori_loop(0, (NCH_S - 1) // 2, chunk2, 0)

            wait_s(msg1_v, s1)
            wait_l(idx0_v, msg0_v, l0)
            counts(idx0_v)
            fire_s(idx0_v, msg0_v, s0)
            wait_s(msg0_v, s0)

            for i in range(NS):
                @pl.when(tid == i)
                def _():
                    pltpu.sync_copy(cnt_v, stage[i])
            plsc.subcore_barrier()

            def zc(r, c2):
                ctot_v[pl.ds(r * LANES, LANES)] = zero16
                return c2
            lax.fori_loop(0, RPT // LANES, zc, 0)
            for i in range(NS):
                pltpu.sync_copy(stage[i].at[pl.ds(rbase, RPT)], tmp_v)

                def acc_cnt(r, c2):
                    s = pl.ds(r * LANES, LANES)
                    ctot_v[s] = ctot_v[s] + tmp_v[s]
                    return c2
                lax.fori_loop(0, RPT // LANES, acc_cnt, 0)

            def rowdiv_at(k):
                def rowdiv(r, c2):
                    iv = jnp.zeros((LANES,), jnp.int32) + (k * RPD + r)
                    c = plsc.load_gather(ctot_v, [iv])
                    invc = 1.0 / jnp.maximum(c, 1.0)
                    for cc in range(D // LANES):
                        s = pl.ds(cc * LANES, LANES)
                        out_v[r, s] = out_v[r, s] * invc
                    return c2
                return rowdiv

            for k in range(NKD):
                rb = rbase + k * RPD

                @pl.when(rb < N_NODES)
                def _():
                    pltpu.sync_copy(acc_sh.at[pl.ds(rb, RPD)], out_v)
                    lax.fori_loop(0, RPD, rowdiv_at(k), 0)
                    pltpu.sync_copy(out_v, out_hbm.at[pl.ds(rb, RPD)])

        @pl.when(core == 0)
        def _():
            side(mu, dst_f, item_out)

        @pl.when(core == 1)
        def _():
            side(mi, src_f, user_out)

    return scatter_kernel


def kernel(user_emb, item_emb, query_table, W_K, b_K, W1, b1, W2, b2,
           edge_index, edge_type, query_idx, userquery_idx, itemquery_idx):
    i32 = jnp.int32
    src = edge_index[0].astype(i32)
    dst = edge_index[1].astype(i32)
    q0 = query_idx[:, 0].astype(i32)
    q1 = query_idx[:, 1].astype(i32)
    uq0 = userquery_idx[:, 0].astype(i32)
    uq1 = userquery_idx[:, 1].astype(i32)
    iq0 = itemquery_idx[:, 0].astype(i32)
    iq1 = itemquery_idx[:, 1].astype(i32)

    mask = (edge_type == 0).astype(jnp.float32)[:, None]
    wargs = (W_K, b_K.reshape(1, D), W1, b1.reshape(1, H), W2,
             b2.reshape(1, D))

    mus, mis = [], []
    for s in range(4):
        gbase, gsz = SPLITS[s], SPLITS[s + 1] - SPLITS[s]
        parts = _build_gather(gbase, gsz)(
            user_emb, item_emb, query_table,
            src, dst, q0, q1, uq0, uq1, iq0, iq1)
        m_u, m_i = _build_dense(gsz)(
            *parts, lax.slice_in_dim(mask, gbase, gbase + gsz), *wargs)
        mus.append(m_u)
        mis.append(m_i)

    user_out, item_out = _build_scatter()(*mus, *mis, dst, src)
    return user_out, item_out


# speedup vs baseline: 1.0123x; 1.0123x over previous
import functools

import jax
import jax.numpy as jnp
from jax import lax
from jax.experimental import pallas as pl
from jax.experimental.pallas import tpu as pltpu
from jax.experimental.pallas import tpu_sc as plsc

N_NODES = 10000
D = 128
E = 160000
H = 256
LANES = 16

NC, NS = 2, 16
NW = NC * NS

SPLITS = (0, 98560, 160000)
NSPL = len(SPLITS) - 1
BG = 40

EPT = E // NS
BS = 80
NCH_S = EPT // BS
RPT = 640
RPD = 80
NKD = RPT // RPD
NPAD = NS * RPT

BE = 1280
NBLK = E // BE


@functools.cache
def _build_gather(gbase, gsz):
    mesh = plsc.VectorSubcoreMesh(core_axis_name="c", subcore_axis_name="s")
    epw = gsz // NW
    nch = epw // BG
    assert epw % BG == 0 and epw % 8 == 0

    @functools.partial(
        pl.kernel,
        mesh=mesh,
        out_type=[jax.ShapeDtypeStruct((gsz, D), jnp.float32)] * 7,
        scratch_types=(
            [pltpu.VMEM((epw,), jnp.int32)] * 8
            + [pltpu.VMEM((BG, D), jnp.float32)] * 16
            + [pltpu.SemaphoreType.DMA] * 4
        ),
        compiler_params=pltpu.CompilerParams(needs_layout_passes=False),
    )
    def gather_kernel(user_hbm, item_hbm, qt_hbm,
                      src_i, dst_i, q0_i, q1_i, uq0_i, uq1_i, iq0_i, iq1_i,
                      o_src, o_dst, o_qs, o_uq0, o_uq1, o_iq0, o_iq1,
                      v_src, v_dst, v_q0, v_q1, v_uq0, v_uq1, v_iq0, v_iq1,
                      *bufs_and_sems):
        rows0 = bufs_and_sems[0:8]
        rows1 = bufs_and_sems[8:16]
        g0, g1, w0, w1 = bufs_and_sems[16:20]
        wid = lax.axis_index("s") * NC + lax.axis_index("c")
        base = wid * epw
        ivs = (v_src, v_dst, v_q0, v_q1, v_uq0, v_uq1, v_iq0, v_iq1)
        ihs = (src_i, dst_i, q0_i, q1_i, uq0_i, uq1_i, iq0_i, iq1_i)
        for iv, ih in zip(ivs, ihs):
            pltpu.async_copy(ih.at[pl.ds(gbase + base, epw)], iv, g1)
        for iv, ih in zip(ivs, ihs):
            pltpu.make_async_copy(ih.at[pl.ds(0, epw)], iv, g1).wait()

        tables = (user_hbm, item_hbm) + (qt_hbm,) * 6
        ohbm = (o_src, o_dst, o_qs, o_uq0, o_uq1, o_iq0, o_iq1)

        def fire_g(j, rows, sem):
            off = j * BG
            for tbl, iv, rv in zip(tables, ivs, rows):
                pltpu.async_copy(tbl.at[iv.at[pl.ds(off, BG)]], rv, sem)

        def wait_g(rows, sem):
            for rv in rows:
                pltpu.make_async_copy(qt_hbm.at[pl.ds(0, BG)], rv, sem).wait()

        def wbufs(rows):
            return (rows[0], rows[1], rows[2], rows[4], rows[5],
                    rows[6], rows[7])

        def fire_w(j, rows, sem):
            ob = base + j * BG
            for ov, rv in zip(ohbm, wbufs(rows)):
                pltpu.async_copy(rv, ov.at[pl.ds(ob, BG)], sem)

        def wait_w(rows, sem):
            for ov, rv in zip(ohbm, wbufs(rows)):
                pltpu.make_async_copy(rv, ov.at[pl.ds(0, BG)], sem).wait()

        def qsum(rows):
            rq0, rq1 = rows[2], rows[3]

            def row(r, c2):
                for c in range(D // LANES):
                    s = pl.ds(c * LANES, LANES)
                    rq0[r, s] = rq0[r, s] + rq1[r, s]
                return c2
            lax.fori_loop(0, BG, row, 0)

        fire_g(0, rows0, g0)
        nbody = (nch - 1) // 2 if nch % 2 == 1 else (nch - 2) // 2

        def body(i, carry):
            a = 2 * i
            b = a + 1

            @pl.when(i > 0)
            def _():
                wait_w(rows1, w1)
            fire_g(b, rows1, g1)
            wait_g(rows0, g0)
            qsum(rows0)
            fire_w(a, rows0, w0)
            wait_w(rows0, w0)
            fire_g(a + 2, rows0, g0)
            wait_g(rows1, g1)
            qsum(rows1)
            fire_w(b, rows1, w1)
            return carry
        lax.fori_loop(0, nbody, body, 0)

        if nch % 2 == 1:
            wait_w(rows1, w1)
            wait_g(rows0, g0)
            qsum(rows0)
            fire_w(nch - 1, rows0, w0)
            wait_w(rows0, w0)
        else:
            wait_w(rows1, w1)
            fire_g(nch - 1, rows1, g1)
            wait_g(rows0, g0)
            qsum(rows0)
            fire_w(nch - 2, rows0, w0)
            wait_g(rows1, g1)
            qsum(rows1)
            fire_w(nch - 1, rows1, w1)
            wait_w(rows0, w0)
            wait_w(rows1, w1)

    return gather_kernel


def _dense_body(se_ref, de_ref, qs_ref, u0_ref, u1_ref, i0_ref, i1_ref,
                mk_ref, wk_ref, bk_ref, w1_ref, b1_ref, w2_ref, b2_ref,
                o1_ref, o2_ref):
    src = se_ref[...]
    dst = de_ref[...]
    qs = qs_ref[...]
    u0 = u0_ref[...]
    u1 = u1_ref[...]
    i0 = i0_ref[...]
    i1 = i1_ref[...]
    mk = mk_ref[...]

    f32 = jnp.float32
    kkey = (jnp.dot(src, wk_ref[0:D, :], preferred_element_type=f32)
            + jnp.dot(u0, wk_ref[D:2 * D, :], preferred_element_type=f32)
            + jnp.dot(u1, wk_ref[2 * D:3 * D, :], preferred_element_type=f32)
            + bk_ref[...])
    inv_sqrt_d = 1.0 / (D ** 0.5)
    s0 = jnp.sum(i0 * kkey, axis=1, keepdims=True) * inv_sqrt_d
    s1 = jnp.sum(i1 * kkey, axis=1, keepdims=True) * inv_sqrt_d
    m = jnp.maximum(s0, s1)
    e0 = jnp.exp(s0 - m)
    e1 = jnp.exp(s1 - m)
    den = e0 + e1
    a0 = e0 / den
    a1 = e1 / den
    iqe = a0 * i0 + a1 * i1
    uqs = u0 + u1

    h = (jnp.dot(src, w1_ref[0:D, :], preferred_element_type=f32)
         + jnp.dot(dst, w1_ref[D:2 * D, :], preferred_element_type=f32)
         + jnp.dot(uqs, w1_ref[2 * D:3 * D, :], preferred_element_type=f32)
         + jnp.dot(iqe, w1_ref[3 * D:4 * D, :], preferred_element_type=f32)
         + b1_ref[...])
    h = jnp.maximum(h, 0.0)
    rec = jnp.dot(h, w2_ref[...], preferred_element_type=f32) + b2_ref[...]
    qe = jnp.where(mk > 0.5, rec, qs)
    o1_ref[...] = src + qe
    o2_ref[...] = dst - qe


@functools.cache
def _build_dense(esz):
    return pl.pallas_call(
        _dense_body,
        grid=(esz // BE,),
        in_specs=[pl.BlockSpec((BE, D), lambda i: (i, 0))] * 7
        + [pl.BlockSpec((BE, 1), lambda i: (i, 0)),
           pl.BlockSpec((3 * D, D), lambda i: (0, 0)),
           pl.BlockSpec((1, D), lambda i: (0, 0)),
           pl.BlockSpec((4 * D, H), lambda i: (0, 0)),
           pl.BlockSpec((1, H), lambda i: (0, 0)),
           pl.BlockSpec((H, D), lambda i: (0, 0)),
           pl.BlockSpec((1, D), lambda i: (0, 0))],
        out_specs=[pl.BlockSpec((BE, D), lambda i: (i, 0))] * 2,
        out_shape=[jax.ShapeDtypeStruct((esz, D), jnp.float32)] * 2,
    )


@functools.cache
def _build_scatter():
    mesh = plsc.VectorSubcoreMesh(core_axis_name="c", subcore_axis_name="s")

    @functools.partial(
        pl.kernel,
        mesh=mesh,
        out_type=[jax.ShapeDtypeStruct((N_NODES, D), jnp.float32),
                  jax.ShapeDtypeStruct((N_NODES, D), jnp.float32)],
        scratch_types=[
            pltpu.VMEM((1, BS), jnp.int32),
            pltpu.VMEM((1, BS), jnp.int32),
            pltpu.VMEM((BS, D), jnp.float32),
            pltpu.VMEM((BS, D), jnp.float32),
            pltpu.VMEM((NPAD,), jnp.float32),
            pltpu.VMEM((RPT,), jnp.float32),
            pltpu.VMEM((RPT,), jnp.float32),
            pltpu.VMEM_SHARED((N_NODES, D), jnp.float32),
        ] + [pltpu.VMEM_SHARED((NPAD,), jnp.float32)] * NS
          + [pltpu.SemaphoreType.DMA] * 4,
        compiler_params=pltpu.CompilerParams(needs_layout_passes=False),
    )
    def scatter_kernel(*refs):
        mu = refs[0:NSPL]
        mi = refs[NSPL:2 * NSPL]
        dst_f, src_f = refs[2 * NSPL:2 * NSPL + 2]
        (user_out, item_out, idx0_v, idx1_v, msg0_v, msg1_v, cnt_v,
         tmp_v, ctot_v, acc_sh) = refs[2 * NSPL + 2:2 * NSPL + 12]
        stage_and_sems = refs[2 * NSPL + 12:]
        out_v = msg0_v
        stage = stage_and_sems[:NS]
        l0, l1, s0, s1 = stage_and_sems[NS:NS + 4]
        core = lax.axis_index("c")
        tid = lax.axis_index("s")
        zero16 = jnp.zeros((LANES,), jnp.float32)
        one16 = jnp.ones((LANES,), jnp.float32)

        def zrow(r, c2):
            for c in range(D // LANES):
                out_v[r, pl.ds(c * LANES, LANES)] = zero16
            return c2
        lax.fori_loop(0, RPD, zrow, 0)

        def zcnt(r, c2):
            cnt_v[pl.ds(r * LANES, LANES)] = zero16
            return c2
        lax.fori_loop(0, NPAD // LANES, zcnt, 0)

        rbase = tid * RPT
        for k in range(NKD):
            @pl.when(rbase + k * RPD < N_NODES)
            def _():
                pltpu.sync_copy(out_v, acc_sh.at[pl.ds(rbase + k * RPD, RPD)])
        plsc.subcore_barrier()

        def side(m_parts, idx_hbm, out_hbm):
            ebase = tid * EPT

            def fire_l(j, iv, mv, sem):
                off = ebase + j * BS
                pltpu.async_copy(idx_hbm.at[pl.ds(off, BS)], iv.at[0], sem)
                for s in range(NSPL):
                    @pl.when((off >= SPLITS[s]) & (off < SPLITS[s + 1]))
                    def _():
                        pltpu.async_copy(
                            m_parts[s].at[pl.ds(off - SPLITS[s], BS)], mv,
                            sem)

            def wait_l(iv, mv, sem):
                pltpu.make_async_copy(idx_hbm.at[pl.ds(0, BS)], iv.at[0],
                                      sem).wait()
                pltpu.make_async_copy(m_parts[0].at[pl.ds(0, BS)], mv,
                                      sem).wait()

            def counts(iv):
                for v in range(BS // LANES):
                    vec = iv[0, pl.ds(v * LANES, LANES)]
                    plsc.addupdate_scatter(cnt_v, [vec], one16)

            def fire_s(iv, mv, sem):
                pltpu.async_copy(mv, acc_sh.at[iv.at[0]], sem, add=True)

            def wait_s(mv, sem):
                pltpu.make_async_copy(m_parts[0].at[pl.ds(0, BS)], mv,
                                      sem).wait()

            fire_l(0, idx0_v, msg0_v, l0)

            def chunk2(i, c2):
                a = 2 * i

                @pl.when(i > 0)
                def _():
                    wait_s(msg1_v, s1)
                fire_l(a + 1, idx1_v, msg1_v, l1)
                wait_l(idx0_v, msg0_v, l0)
                counts(idx0_v)
                fire_s(idx0_v, msg0_v, s0)
                wait_s(msg0_v, s0)
                fire_l(a + 2, idx0_v, msg0_v, l0)
                wait_l(idx1_v, msg1_v, l1)
                counts(idx1_v)
                fire_s(idx1_v, msg1_v, s1)
                return c2
            lax.fori_loop(0, (NCH_S - 1) // 2, chunk2, 0)

            wait_s(msg1_v, s1)
            wait_l(idx0_v, msg0_v, l0)
            counts(idx0_v)
            fire_s(idx0_v, msg0_v, s0)
            wait_s(msg0_v, s0)

            for i in range(NS):
                @pl.when(tid == i)
                def _():
                    pltpu.sync_copy(cnt_v, stage[i])
            plsc.subcore_barrier()

            def zc(r, c2):
                ctot_v[pl.ds(r * LANES, LANES)] = zero16
                return c2
            lax.fori_loop(0, RPT // LANES, zc, 0)
            for i in range(NS):
                pltpu.sync_copy(stage[i].at[pl.ds(rbase, RPT)], tmp_v)

                def acc_cnt(r, c2):
                    s = pl.ds(r * LANES, LANES)
                    ctot_v[s] = ctot_v[s] + tmp_v[s]
                    return c2
                lax.fori_loop(0, RPT // LANES, acc_cnt, 0)

            def rowdiv_at(k):
                def rowdiv(r, c2):
                    iv = jnp.zeros((LANES,), jnp.int32) + (k * RPD + r)
                    c = plsc.load_gather(ctot_v, [iv])
                    invc = 1.0 / jnp.maximum(c, 1.0)
                    for cc in range(D // LANES):
                        s = pl.ds(cc * LANES, LANES)
                        out_v[r, s] = out_v[r, s] * invc
                    return c2
                return rowdiv

            for k in range(NKD):
                rb = rbase + k * RPD

                @pl.when(rb < N_NODES)
                def _():
                    pltpu.sync_copy(acc_sh.at[pl.ds(rb, RPD)], out_v)
                    lax.fori_loop(0, RPD, rowdiv_at(k), 0)
                    pltpu.sync_copy(out_v, out_hbm.at[pl.ds(rb, RPD)])

        @pl.when(core == 0)
        def _():
            side(mu, dst_f, item_out)

        @pl.when(core == 1)
        def _():
            side(mi, src_f, user_out)

    return scatter_kernel


def kernel(user_emb, item_emb, query_table, W_K, b_K, W1, b1, W2, b2,
           edge_index, edge_type, query_idx, userquery_idx, itemquery_idx):
    i32 = jnp.int32
    src = edge_index[0].astype(i32)
    dst = edge_index[1].astype(i32)
    q0 = query_idx[:, 0].astype(i32)
    q1 = query_idx[:, 1].astype(i32)
    uq0 = userquery_idx[:, 0].astype(i32)
    uq1 = userquery_idx[:, 1].astype(i32)
    iq0 = itemquery_idx[:, 0].astype(i32)
    iq1 = itemquery_idx[:, 1].astype(i32)

    mask = (edge_type == 0).astype(jnp.float32)[:, None]
    wargs = (W_K, b_K.reshape(1, D), W1, b1.reshape(1, H), W2,
             b2.reshape(1, D))

    mus, mis = [], []
    for s in range(NSPL):
        gbase, gsz = SPLITS[s], SPLITS[s + 1] - SPLITS[s]
        parts = _build_gather(gbase, gsz)(
            user_emb, item_emb, query_table,
            src, dst, q0, q1, uq0, uq1, iq0, iq1)
        m_u, m_i = _build_dense(gsz)(
            *parts, lax.slice_in_dim(mask, gbase, gbase + gsz), *wargs)
        mus.append(m_u)
        mis.append(m_i)

    user_out, item_out = _build_scatter()(*mus, *mis, dst, src)
    return user_out, item_out


# speedup vs baseline: 1.0375x; 1.0249x over previous
import functools

import jax
import jax.numpy as jnp
from jax import lax
from jax.experimental import pallas as pl
from jax.experimental.pallas import tpu as pltpu
from jax.experimental.pallas import tpu_sc as plsc

N_NODES = 10000
D = 128
E = 160000
H = 256
LANES = 16

NC, NS = 2, 16
NW = NC * NS

SPLITS = (0, 160000)
NSPL = len(SPLITS) - 1
BG = 40

EPT = E // NS
BS = 80
NCH_S = EPT // BS
RPT = 640
RPD = 80
NKD = RPT // RPD
NPAD = NS * RPT

BE = 1280
NBLK = E // BE


@functools.cache
def _build_gather(gbase, gsz):
    mesh = plsc.VectorSubcoreMesh(core_axis_name="c", subcore_axis_name="s")
    epw = gsz // NW
    nch = epw // BG
    assert epw % BG == 0 and epw % 8 == 0

    @functools.partial(
        pl.kernel,
        mesh=mesh,
        out_type=[jax.ShapeDtypeStruct((gsz, D), jnp.float32)] * 7,
        scratch_types=(
            [pltpu.VMEM((epw,), jnp.int32)] * 8
            + [pltpu.VMEM((BG, D), jnp.float32)] * 16
            + [pltpu.SemaphoreType.DMA] * 4
        ),
        compiler_params=pltpu.CompilerParams(needs_layout_passes=False),
    )
    def gather_kernel(user_hbm, item_hbm, qt_hbm,
                      src_i, dst_i, q0_i, q1_i, uq0_i, uq1_i, iq0_i, iq1_i,
                      o_src, o_dst, o_qs, o_uq0, o_uq1, o_iq0, o_iq1,
                      v_src, v_dst, v_q0, v_q1, v_uq0, v_uq1, v_iq0, v_iq1,
                      *bufs_and_sems):
        rows0 = bufs_and_sems[0:8]
        rows1 = bufs_and_sems[8:16]
        g0, g1, w0, w1 = bufs_and_sems[16:20]
        wid = lax.axis_index("s") * NC + lax.axis_index("c")
        base = wid * epw
        ivs = (v_src, v_dst, v_q0, v_q1, v_uq0, v_uq1, v_iq0, v_iq1)
        ihs = (src_i, dst_i, q0_i, q1_i, uq0_i, uq1_i, iq0_i, iq1_i)
        for iv, ih in zip(ivs, ihs):
            pltpu.async_copy(ih.at[pl.ds(gbase + base, epw)], iv, g1)
        for iv, ih in zip(ivs, ihs):
            pltpu.make_async_copy(ih.at[pl.ds(0, epw)], iv, g1).wait()

        tables = (user_hbm, item_hbm) + (qt_hbm,) * 6
        ohbm = (o_src, o_dst, o_qs, o_uq0, o_uq1, o_iq0, o_iq1)

        def fire_g(j, rows, sem):
            off = j * BG
            for tbl, iv, rv in zip(tables, ivs, rows):
                pltpu.async_copy(tbl.at[iv.at[pl.ds(off, BG)]], rv, sem)

        def wait_g(rows, sem):
            for rv in rows:
                pltpu.make_async_copy(qt_hbm.at[pl.ds(0, BG)], rv, sem).wait()

        def wbufs(rows):
            return (rows[0], rows[1], rows[2], rows[4], rows[5],
                    rows[6], rows[7])

        def fire_w(j, rows, sem):
            ob = base + j * BG
            for ov, rv in zip(ohbm, wbufs(rows)):
                pltpu.async_copy(rv, ov.at[pl.ds(ob, BG)], sem)

        def wait_w(rows, sem):
            for ov, rv in zip(ohbm, wbufs(rows)):
                pltpu.make_async_copy(rv, ov.at[pl.ds(0, BG)], sem).wait()

        def qsum(rows):
            rq0, rq1 = rows[2], rows[3]

            def row(r, c2):
                for c in range(D // LANES):
                    s = pl.ds(c * LANES, LANES)
                    rq0[r, s] = rq0[r, s] + rq1[r, s]
                return c2
            lax.fori_loop(0, BG, row, 0)

        fire_g(0, rows0, g0)
        nbody = (nch - 1) // 2 if nch % 2 == 1 else (nch - 2) // 2

        def body(i, carry):
            a = 2 * i
            b = a + 1

            @pl.when(i > 0)
            def _():
                wait_w(rows1, w1)
            fire_g(b, rows1, g1)
            wait_g(rows0, g0)
            qsum(rows0)
            fire_w(a, rows0, w0)
            wait_w(rows0, w0)
            fire_g(a + 2, rows0, g0)
            wait_g(rows1, g1)
            qsum(rows1)
            fire_w(b, rows1, w1)
            return carry
        lax.fori_loop(0, nbody, body, 0)

        if nch % 2 == 1:
            wait_w(rows1, w1)
            wait_g(rows0, g0)
            qsum(rows0)
            fire_w(nch - 1, rows0, w0)
            wait_w(rows0, w0)
        else:
            wait_w(rows1, w1)
            fire_g(nch - 1, rows1, g1)
            wait_g(rows0, g0)
            qsum(rows0)
            fire_w(nch - 2, rows0, w0)
            wait_g(rows1, g1)
            qsum(rows1)
            fire_w(nch - 1, rows1, w1)
            wait_w(rows0, w0)
            wait_w(rows1, w1)

    return gather_kernel


def _dense_body(se_ref, de_ref, qs_ref, u0_ref, u1_ref, i0_ref, i1_ref,
                mk_ref, wk_ref, bk_ref, w1_ref, b1_ref, w2_ref, b2_ref,
                o1_ref, o2_ref):
    src = se_ref[...]
    dst = de_ref[...]
    qs = qs_ref[...]
    u0 = u0_ref[...]
    u1 = u1_ref[...]
    i0 = i0_ref[...]
    i1 = i1_ref[...]
    mk = mk_ref[...]

    f32 = jnp.float32
    kkey = (jnp.dot(src, wk_ref[0:D, :], preferred_element_type=f32)
            + jnp.dot(u0, wk_ref[D:2 * D, :], preferred_element_type=f32)
            + jnp.dot(u1, wk_ref[2 * D:3 * D, :], preferred_element_type=f32)
            + bk_ref[...])
    inv_sqrt_d = 1.0 / (D ** 0.5)
    s0 = jnp.sum(i0 * kkey, axis=1, keepdims=True) * inv_sqrt_d
    s1 = jnp.sum(i1 * kkey, axis=1, keepdims=True) * inv_sqrt_d
    m = jnp.maximum(s0, s1)
    e0 = jnp.exp(s0 - m)
    e1 = jnp.exp(s1 - m)
    den = e0 + e1
    a0 = e0 / den
    a1 = e1 / den
    iqe = a0 * i0 + a1 * i1
    uqs = u0 + u1

    h = (jnp.dot(src, w1_ref[0:D, :], preferred_element_type=f32)
         + jnp.dot(dst, w1_ref[D:2 * D, :], preferred_element_type=f32)
         + jnp.dot(uqs, w1_ref[2 * D:3 * D, :], preferred_element_type=f32)
         + jnp.dot(iqe, w1_ref[3 * D:4 * D, :], preferred_element_type=f32)
         + b1_ref[...])
    h = jnp.maximum(h, 0.0)
    rec = jnp.dot(h, w2_ref[...], preferred_element_type=f32) + b2_ref[...]
    qe = jnp.where(mk > 0.5, rec, qs)
    o1_ref[...] = src + qe
    o2_ref[...] = dst - qe


@functools.cache
def _build_dense(esz):
    return pl.pallas_call(
        _dense_body,
        grid=(esz // BE,),
        in_specs=[pl.BlockSpec((BE, D), lambda i: (i, 0))] * 7
        + [pl.BlockSpec((BE, 1), lambda i: (i, 0)),
           pl.BlockSpec((3 * D, D), lambda i: (0, 0)),
           pl.BlockSpec((1, D), lambda i: (0, 0)),
           pl.BlockSpec((4 * D, H), lambda i: (0, 0)),
           pl.BlockSpec((1, H), lambda i: (0, 0)),
           pl.BlockSpec((H, D), lambda i: (0, 0)),
           pl.BlockSpec((1, D), lambda i: (0, 0))],
        out_specs=[pl.BlockSpec((BE, D), lambda i: (i, 0))] * 2,
        out_shape=[jax.ShapeDtypeStruct((esz, D), jnp.float32)] * 2,
    )


@functools.cache
def _build_scatter():
    mesh = plsc.VectorSubcoreMesh(core_axis_name="c", subcore_axis_name="s")

    @functools.partial(
        pl.kernel,
        mesh=mesh,
        out_type=[jax.ShapeDtypeStruct((N_NODES, D), jnp.float32),
                  jax.ShapeDtypeStruct((N_NODES, D), jnp.float32)],
        scratch_types=[
            pltpu.VMEM((1, BS), jnp.int32),
            pltpu.VMEM((1, BS), jnp.int32),
            pltpu.VMEM((BS, D), jnp.float32),
            pltpu.VMEM((BS, D), jnp.float32),
            pltpu.VMEM((NPAD,), jnp.float32),
            pltpu.VMEM((RPT,), jnp.float32),
            pltpu.VMEM((RPT,), jnp.float32),
            pltpu.VMEM_SHARED((N_NODES, D), jnp.float32),
        ] + [pltpu.VMEM_SHARED((NPAD,), jnp.float32)] * NS
          + [pltpu.SemaphoreType.DMA] * 4,
        compiler_params=pltpu.CompilerParams(needs_layout_passes=False),
    )
    def scatter_kernel(*refs):
        mu = refs[0:NSPL]
        mi = refs[NSPL:2 * NSPL]
        dst_f, src_f = refs[2 * NSPL:2 * NSPL + 2]
        (user_out, item_out, idx0_v, idx1_v, msg0_v, msg1_v, cnt_v,
         tmp_v, ctot_v, acc_sh) = refs[2 * NSPL + 2:2 * NSPL + 12]
        stage_and_sems = refs[2 * NSPL + 12:]
        out_v = msg0_v
        stage = stage_and_sems[:NS]
        l0, l1, s0, s1 = stage_and_sems[NS:NS + 4]
        core = lax.axis_index("c")
        tid = lax.axis_index("s")
        zero16 = jnp.zeros((LANES,), jnp.float32)
        one16 = jnp.ones((LANES,), jnp.float32)

        def zrow(r, c2):
            for c in range(D // LANES):
                out_v[r, pl.ds(c * LANES, LANES)] = zero16
            return c2
        lax.fori_loop(0, RPD, zrow, 0)

        def zcnt(r, c2):
            cnt_v[pl.ds(r * LANES, LANES)] = zero16
            return c2
        lax.fori_loop(0, NPAD // LANES, zcnt, 0)

        rbase = tid * RPT
        for k in range(NKD):
            @pl.when(rbase + k * RPD < N_NODES)
            def _():
                pltpu.sync_copy(out_v, acc_sh.at[pl.ds(rbase + k * RPD, RPD)])
        plsc.subcore_barrier()

        def side(m_parts, idx_hbm, out_hbm):
            ebase = tid * EPT

            def fire_l(j, iv, mv, sem):
                off = ebase + j * BS
                pltpu.async_copy(idx_hbm.at[pl.ds(off, BS)], iv.at[0], sem)
                for s in range(NSPL):
                    @pl.when((off >= SPLITS[s]) & (off < SPLITS[s + 1]))
                    def _():
                        pltpu.async_copy(
                            m_parts[s].at[pl.ds(off - SPLITS[s], BS)], mv,
                            sem)

            def wait_l(iv, mv, sem):
                pltpu.make_async_copy(idx_hbm.at[pl.ds(0, BS)], iv.at[0],
                                      sem).wait()
                pltpu.make_async_copy(m_parts[0].at[pl.ds(0, BS)], mv,
                                      sem).wait()

            def counts(iv):
                for v in range(BS // LANES):
                    vec = iv[0, pl.ds(v * LANES, LANES)]
                    plsc.addupdate_scatter(cnt_v, [vec], one16)

            def fire_s(iv, mv, sem):
                pltpu.async_copy(mv, acc_sh.at[iv.at[0]], sem, add=True)

            def wait_s(mv, sem):
                pltpu.make_async_copy(m_parts[0].at[pl.ds(0, BS)], mv,
                                      sem).wait()

            fire_l(0, idx0_v, msg0_v, l0)

            def chunk2(i, c2):
                a = 2 * i

                @pl.when(i > 0)
                def _():
                    wait_s(msg1_v, s1)
                fire_l(a + 1, idx1_v, msg1_v, l1)
                wait_l(idx0_v, msg0_v, l0)
                counts(idx0_v)
                fire_s(idx0_v, msg0_v, s0)
                wait_s(msg0_v, s0)
                fire_l(a + 2, idx0_v, msg0_v, l0)
                wait_l(idx1_v, msg1_v, l1)
                counts(idx1_v)
                fire_s(idx1_v, msg1_v, s1)
                return c2
            lax.fori_loop(0, (NCH_S - 1) // 2, chunk2, 0)

            wait_s(msg1_v, s1)
            wait_l(idx0_v, msg0_v, l0)
            counts(idx0_v)
            fire_s(idx0_v, msg0_v, s0)
            wait_s(msg0_v, s0)

            for i in range(NS):
                @pl.when(tid == i)
                def _():
                    pltpu.sync_copy(cnt_v, stage[i])
            plsc.subcore_barrier()

            def zc(r, c2):
                ctot_v[pl.ds(r * LANES, LANES)] = zero16
                return c2
            lax.fori_loop(0, RPT // LANES, zc, 0)
            for i in range(NS):
                pltpu.sync_copy(stage[i].at[pl.ds(rbase, RPT)], tmp_v)

                def acc_cnt(r, c2):
                    s = pl.ds(r * LANES, LANES)
                    ctot_v[s] = ctot_v[s] + tmp_v[s]
                    return c2
                lax.fori_loop(0, RPT // LANES, acc_cnt, 0)

            def rowdiv_at(k):
                def rowdiv(r, c2):
                    iv = jnp.zeros((LANES,), jnp.int32) + (k * RPD + r)
                    c = plsc.load_gather(ctot_v, [iv])
                    invc = 1.0 / jnp.maximum(c, 1.0)
                    for cc in range(D // LANES):
                        s = pl.ds(cc * LANES, LANES)
                        out_v[r, s] = out_v[r, s] * invc
                    return c2
                return rowdiv

            for k in range(NKD):
                rb = rbase + k * RPD

                @pl.when(rb < N_NODES)
                def _():
                    pltpu.sync_copy(acc_sh.at[pl.ds(rb, RPD)], out_v)
                    lax.fori_loop(0, RPD, rowdiv_at(k), 0)
                    pltpu.sync_copy(out_v, out_hbm.at[pl.ds(rb, RPD)])

        @pl.when(core == 0)
        def _():
            side(mu, dst_f, item_out)

        @pl.when(core == 1)
        def _():
            side(mi, src_f, user_out)

    return scatter_kernel


def kernel(user_emb, item_emb, query_table, W_K, b_K, W1, b1, W2, b2,
           edge_index, edge_type, query_idx, userquery_idx, itemquery_idx):
    i32 = jnp.int32
    src = edge_index[0].astype(i32)
    dst = edge_index[1].astype(i32)
    q0 = query_idx[:, 0].astype(i32)
    q1 = query_idx[:, 1].astype(i32)
    uq0 = userquery_idx[:, 0].astype(i32)
    uq1 = userquery_idx[:, 1].astype(i32)
    iq0 = itemquery_idx[:, 0].astype(i32)
    iq1 = itemquery_idx[:, 1].astype(i32)

    mask = (edge_type == 0).astype(jnp.float32)[:, None]
    wargs = (W_K, b_K.reshape(1, D), W1, b1.reshape(1, H), W2,
             b2.reshape(1, D))

    mus, mis = [], []
    for s in range(NSPL):
        gbase, gsz = SPLITS[s], SPLITS[s + 1] - SPLITS[s]
        parts = _build_gather(gbase, gsz)(
            user_emb, item_emb, query_table,
            src, dst, q0, q1, uq0, uq1, iq0, iq1)
        m_u, m_i = _build_dense(gsz)(
            *parts, lax.slice_in_dim(mask, gbase, gbase + gsz), *wargs)
        mus.append(m_u)
        mis.append(m_i)

    user_out, item_out = _build_scatter()(*mus, *mis, dst, src)
    return user_out, item_out


# speedup vs baseline: 1.0641x; 1.0256x over previous
import functools

import jax
import jax.numpy as jnp
from jax import lax
from jax.experimental import pallas as pl
from jax.experimental.pallas import tpu as pltpu
from jax.experimental.pallas import tpu_sc as plsc

N_NODES = 10000
D = 128
E = 160000
H = 256
LANES = 16

NC, NS = 2, 16
NW = NC * NS

SPLITS = (0, 160000)
NSPL = len(SPLITS) - 1
BG = 40

EPT = E // NS
BS = 80
NCH_S = EPT // BS
RPT = 640
RPD = 80
NKD = RPT // RPD
NPAD = NS * RPT

BE = 3200
NBLK = E // BE


@functools.cache
def _build_gather(gbase, gsz):
    mesh = plsc.VectorSubcoreMesh(core_axis_name="c", subcore_axis_name="s")
    epw = gsz // NW
    nch = epw // BG
    assert epw % BG == 0 and epw % 8 == 0

    @functools.partial(
        pl.kernel,
        mesh=mesh,
        out_type=[jax.ShapeDtypeStruct((gsz, D), jnp.float32)] * 7,
        scratch_types=(
            [pltpu.VMEM((epw,), jnp.int32)] * 8
            + [pltpu.VMEM((BG, D), jnp.float32)] * 16
            + [pltpu.SemaphoreType.DMA] * 4
        ),
        compiler_params=pltpu.CompilerParams(needs_layout_passes=False),
    )
    def gather_kernel(user_hbm, item_hbm, qt_hbm,
                      src_i, dst_i, q0_i, q1_i, uq0_i, uq1_i, iq0_i, iq1_i,
                      o_src, o_dst, o_qs, o_uq0, o_uq1, o_iq0, o_iq1,
                      v_src, v_dst, v_q0, v_q1, v_uq0, v_uq1, v_iq0, v_iq1,
                      *bufs_and_sems):
        rows0 = bufs_and_sems[0:8]
        rows1 = bufs_and_sems[8:16]
        g0, g1, w0, w1 = bufs_and_sems[16:20]
        wid = lax.axis_index("s") * NC + lax.axis_index("c")
        base = wid * epw
        ivs = (v_src, v_dst, v_q0, v_q1, v_uq0, v_uq1, v_iq0, v_iq1)
        ihs = (src_i, dst_i, q0_i, q1_i, uq0_i, uq1_i, iq0_i, iq1_i)
        for iv, ih in zip(ivs, ihs):
            pltpu.async_copy(ih.at[pl.ds(gbase + base, epw)], iv, g1)
        for iv, ih in zip(ivs, ihs):
            pltpu.make_async_copy(ih.at[pl.ds(0, epw)], iv, g1).wait()

        tables = (user_hbm, item_hbm) + (qt_hbm,) * 6
        ohbm = (o_src, o_dst, o_qs, o_uq0, o_uq1, o_iq0, o_iq1)

        def fire_g(j, rows, sem):
            off = j * BG
            for tbl, iv, rv in zip(tables, ivs, rows):
                pltpu.async_copy(tbl.at[iv.at[pl.ds(off, BG)]], rv, sem)

        def wait_g(rows, sem):
            for rv in rows:
                pltpu.make_async_copy(qt_hbm.at[pl.ds(0, BG)], rv, sem).wait()

        def wbufs(rows):
            return (rows[0], rows[1], rows[2], rows[4], rows[5],
                    rows[6], rows[7])

        def fire_w(j, rows, sem):
            ob = base + j * BG
            for ov, rv in zip(ohbm, wbufs(rows)):
                pltpu.async_copy(rv, ov.at[pl.ds(ob, BG)], sem)

        def wait_w(rows, sem):
            for ov, rv in zip(ohbm, wbufs(rows)):
                pltpu.make_async_copy(rv, ov.at[pl.ds(0, BG)], sem).wait()

        def qsum(rows):
            rq0, rq1 = rows[2], rows[3]

            def row(r, c2):
                for c in range(D // LANES):
                    s = pl.ds(c * LANES, LANES)
                    rq0[r, s] = rq0[r, s] + rq1[r, s]
                return c2
            lax.fori_loop(0, BG, row, 0)

        fire_g(0, rows0, g0)
        nbody = (nch - 1) // 2 if nch % 2 == 1 else (nch - 2) // 2

        def body(i, carry):
            a = 2 * i
            b = a + 1

            @pl.when(i > 0)
            def _():
                wait_w(rows1, w1)
            fire_g(b, rows1, g1)
            wait_g(rows0, g0)
            qsum(rows0)
            fire_w(a, rows0, w0)
            wait_w(rows0, w0)
            fire_g(a + 2, rows0, g0)
            wait_g(rows1, g1)
            qsum(rows1)
            fire_w(b, rows1, w1)
            return carry
        lax.fori_loop(0, nbody, body, 0)

        if nch % 2 == 1:
            wait_w(rows1, w1)
            wait_g(rows0, g0)
            qsum(rows0)
            fire_w(nch - 1, rows0, w0)
            wait_w(rows0, w0)
        else:
            wait_w(rows1, w1)
            fire_g(nch - 1, rows1, g1)
            wait_g(rows0, g0)
            qsum(rows0)
            fire_w(nch - 2, rows0, w0)
            wait_g(rows1, g1)
            qsum(rows1)
            fire_w(nch - 1, rows1, w1)
            wait_w(rows0, w0)
            wait_w(rows1, w1)

    return gather_kernel


def _dense_body(se_ref, de_ref, qs_ref, u0_ref, u1_ref, i0_ref, i1_ref,
                mk_ref, wk_ref, bk_ref, w1_ref, b1_ref, w2_ref, b2_ref,
                o1_ref, o2_ref):
    src = se_ref[...]
    dst = de_ref[...]
    qs = qs_ref[...]
    u0 = u0_ref[...]
    u1 = u1_ref[...]
    i0 = i0_ref[...]
    i1 = i1_ref[...]
    mk = mk_ref[...]

    f32 = jnp.float32
    kkey = (jnp.dot(src, wk_ref[0:D, :], preferred_element_type=f32)
            + jnp.dot(u0, wk_ref[D:2 * D, :], preferred_element_type=f32)
            + jnp.dot(u1, wk_ref[2 * D:3 * D, :], preferred_element_type=f32)
            + bk_ref[...])
    inv_sqrt_d = 1.0 / (D ** 0.5)
    s0 = jnp.sum(i0 * kkey, axis=1, keepdims=True) * inv_sqrt_d
    s1 = jnp.sum(i1 * kkey, axis=1, keepdims=True) * inv_sqrt_d
    m = jnp.maximum(s0, s1)
    e0 = jnp.exp(s0 - m)
    e1 = jnp.exp(s1 - m)
    den = e0 + e1
    a0 = e0 / den
    a1 = e1 / den
    iqe = a0 * i0 + a1 * i1
    uqs = u0 + u1

    h = (jnp.dot(src, w1_ref[0:D, :], preferred_element_type=f32)
         + jnp.dot(dst, w1_ref[D:2 * D, :], preferred_element_type=f32)
         + jnp.dot(uqs, w1_ref[2 * D:3 * D, :], preferred_element_type=f32)
         + jnp.dot(iqe, w1_ref[3 * D:4 * D, :], preferred_element_type=f32)
         + b1_ref[...])
    h = jnp.maximum(h, 0.0)
    rec = jnp.dot(h, w2_ref[...], preferred_element_type=f32) + b2_ref[...]
    qe = jnp.where(mk > 0.5, rec, qs)
    o1_ref[...] = src + qe
    o2_ref[...] = dst - qe


@functools.cache
def _build_dense(esz):
    return pl.pallas_call(
        _dense_body,
        grid=(esz // BE,),
        in_specs=[pl.BlockSpec((BE, D), lambda i: (i, 0))] * 7
        + [pl.BlockSpec((BE, 1), lambda i: (i, 0)),
           pl.BlockSpec((3 * D, D), lambda i: (0, 0)),
           pl.BlockSpec((1, D), lambda i: (0, 0)),
           pl.BlockSpec((4 * D, H), lambda i: (0, 0)),
           pl.BlockSpec((1, H), lambda i: (0, 0)),
           pl.BlockSpec((H, D), lambda i: (0, 0)),
           pl.BlockSpec((1, D), lambda i: (0, 0))],
        out_specs=[pl.BlockSpec((BE, D), lambda i: (i, 0))] * 2,
        out_shape=[jax.ShapeDtypeStruct((esz, D), jnp.float32)] * 2,
    )


@functools.cache
def _build_scatter():
    mesh = plsc.VectorSubcoreMesh(core_axis_name="c", subcore_axis_name="s")

    @functools.partial(
        pl.kernel,
        mesh=mesh,
        out_type=[jax.ShapeDtypeStruct((N_NODES, D), jnp.float32),
                  jax.ShapeDtypeStruct((N_NODES, D), jnp.float32)],
        scratch_types=[
            pltpu.VMEM((1, BS), jnp.int32),
            pltpu.VMEM((1, BS), jnp.int32),
            pltpu.VMEM((BS, D), jnp.float32),
            pltpu.VMEM((BS, D), jnp.float32),
            pltpu.VMEM((NPAD,), jnp.float32),
            pltpu.VMEM((RPT,), jnp.float32),
            pltpu.VMEM((RPT,), jnp.float32),
            pltpu.VMEM_SHARED((N_NODES, D), jnp.float32),
        ] + [pltpu.VMEM_SHARED((NPAD,), jnp.float32)] * NS
          + [pltpu.SemaphoreType.DMA] * 4,
        compiler_params=pltpu.CompilerParams(needs_layout_passes=False),
    )
    def scatter_kernel(*refs):
        mu = refs[0:NSPL]
        mi = refs[NSPL:2 * NSPL]
        dst_f, src_f = refs[2 * NSPL:2 * NSPL + 2]
        (user_out, item_out, idx0_v, idx1_v, msg0_v, msg1_v, cnt_v,
         tmp_v, ctot_v, acc_sh) = refs[2 * NSPL + 2:2 * NSPL + 12]
        stage_and_sems = refs[2 * NSPL + 12:]
        out_v = msg0_v
        stage = stage_and_sems[:NS]
        l0, l1, s0, s1 = stage_and_sems[NS:NS + 4]
        core = lax.axis_index("c")
        tid = lax.axis_index("s")
        zero16 = jnp.zeros((LANES,), jnp.float32)
        one16 = jnp.ones((LANES,), jnp.float32)

        def zrow(r, c2):
            for c in range(D // LANES):
                out_v[r, pl.ds(c * LANES, LANES)] = zero16
            return c2
        lax.fori_loop(0, RPD, zrow, 0)

        def zcnt(r, c2):
            cnt_v[pl.ds(r * LANES, LANES)] = zero16
            return c2
        lax.fori_loop(0, NPAD // LANES, zcnt, 0)

        rbase = tid * RPT
        for k in range(NKD):
            @pl.when(rbase + k * RPD < N_NODES)
            def _():
                pltpu.sync_copy(out_v, acc_sh.at[pl.ds(rbase + k * RPD, RPD)])
        plsc.subcore_barrier()

        def side(m_parts, idx_hbm, out_hbm):
            ebase = tid * EPT

            def fire_l(j, iv, mv, sem):
                off = ebase + j * BS
                pltpu.async_copy(idx_hbm.at[pl.ds(off, BS)], iv.at[0], sem)
                for s in range(NSPL):
                    @pl.when((off >= SPLITS[s]) & (off < SPLITS[s + 1]))
                    def _():
                        pltpu.async_copy(
                            m_parts[s].at[pl.ds(off - SPLITS[s], BS)], mv,
                            sem)

            def wait_l(iv, mv, sem):
                pltpu.make_async_copy(idx_hbm.at[pl.ds(0, BS)], iv.at[0],
                                      sem).wait()
                pltpu.make_async_copy(m_parts[0].at[pl.ds(0, BS)], mv,
                                      sem).wait()

            def counts(iv):
                for v in range(BS // LANES):
                    vec = iv[0, pl.ds(v * LANES, LANES)]
                    plsc.addupdate_scatter(cnt_v, [vec], one16)

            def fire_s(iv, mv, sem):
                pltpu.async_copy(mv, acc_sh.at[iv.at[0]], sem, add=True)

            def wait_s(mv, sem):
                pltpu.make_async_copy(m_parts[0].at[pl.ds(0, BS)], mv,
                                      sem).wait()

            fire_l(0, idx0_v, msg0_v, l0)

            def chunk2(i, c2):
                a = 2 * i

                @pl.when(i > 0)
                def _():
                    wait_s(msg1_v, s1)
                fire_l(a + 1, idx1_v, msg1_v, l1)
                wait_l(idx0_v, msg0_v, l0)
                counts(idx0_v)
                fire_s(idx0_v, msg0_v, s0)
                wait_s(msg0_v, s0)
                fire_l(a + 2, idx0_v, msg0_v, l0)
                wait_l(idx1_v, msg1_v, l1)
                counts(idx1_v)
                fire_s(idx1_v, msg1_v, s1)
                return c2
            lax.fori_loop(0, (NCH_S - 1) // 2, chunk2, 0)

            wait_s(msg1_v, s1)
            wait_l(idx0_v, msg0_v, l0)
            counts(idx0_v)
            fire_s(idx0_v, msg0_v, s0)
            wait_s(msg0_v, s0)

            for i in range(NS):
                @pl.when(tid == i)
                def _():
                    pltpu.sync_copy(cnt_v, stage[i])
            plsc.subcore_barrier()

            def zc(r, c2):
                ctot_v[pl.ds(r * LANES, LANES)] = zero16
                return c2
            lax.fori_loop(0, RPT // LANES, zc, 0)
            for i in range(NS):
                pltpu.sync_copy(stage[i].at[pl.ds(rbase, RPT)], tmp_v)

                def acc_cnt(r, c2):
                    s = pl.ds(r * LANES, LANES)
                    ctot_v[s] = ctot_v[s] + tmp_v[s]
                    return c2
                lax.fori_loop(0, RPT // LANES, acc_cnt, 0)

            def rowdiv_at(k):
                def rowdiv(r, c2):
                    iv = jnp.zeros((LANES,), jnp.int32) + (k * RPD + r)
                    c = plsc.load_gather(ctot_v, [iv])
                    invc = 1.0 / jnp.maximum(c, 1.0)
                    for cc in range(D // LANES):
                        s = pl.ds(cc * LANES, LANES)
                        out_v[r, s] = out_v[r, s] * invc
                    return c2
                return rowdiv

            for k in range(NKD):
                rb = rbase + k * RPD

                @pl.when(rb < N_NODES)
                def _():
                    pltpu.sync_copy(acc_sh.at[pl.ds(rb, RPD)], out_v)
                    lax.fori_loop(0, RPD, rowdiv_at(k), 0)
                    pltpu.sync_copy(out_v, out_hbm.at[pl.ds(rb, RPD)])

        @pl.when(core == 0)
        def _():
            side(mu, dst_f, item_out)

        @pl.when(core == 1)
        def _():
            side(mi, src_f, user_out)

    return scatter_kernel


def kernel(user_emb, item_emb, query_table, W_K, b_K, W1, b1, W2, b2,
           edge_index, edge_type, query_idx, userquery_idx, itemquery_idx):
    i32 = jnp.int32
    src = edge_index[0].astype(i32)
    dst = edge_index[1].astype(i32)
    q0 = query_idx[:, 0].astype(i32)
    q1 = query_idx[:, 1].astype(i32)
    uq0 = userquery_idx[:, 0].astype(i32)
    uq1 = userquery_idx[:, 1].astype(i32)
    iq0 = itemquery_idx[:, 0].astype(i32)
    iq1 = itemquery_idx[:, 1].astype(i32)

    mask = (edge_type == 0).astype(jnp.float32)[:, None]
    wargs = (W_K, b_K.reshape(1, D), W1, b1.reshape(1, H), W2,
             b2.reshape(1, D))

    mus, mis = [], []
    for s in range(NSPL):
        gbase, gsz = SPLITS[s], SPLITS[s + 1] - SPLITS[s]
        parts = _build_gather(gbase, gsz)(
            user_emb, item_emb, query_table,
            src, dst, q0, q1, uq0, uq1, iq0, iq1)
        m_u, m_i = _build_dense(gsz)(
            *parts, lax.slice_in_dim(mask, gbase, gbase + gsz), *wargs)
        mus.append(m_u)
        mis.append(m_i)

    user_out, item_out = _build_scatter()(*mus, *mis, dst, src)
    return user_out, item_out


# speedup vs baseline: 1.0681x; 1.0037x over previous
import functools

import jax
import jax.numpy as jnp
from jax import lax
from jax.experimental import pallas as pl
from jax.experimental.pallas import tpu as pltpu
from jax.experimental.pallas import tpu_sc as plsc

N_NODES = 10000
D = 128
E = 160000
H = 256
LANES = 16

NC, NS = 2, 16
NW = NC * NS

SPLITS = (0, 160000)
NSPL = len(SPLITS) - 1
BG = 40

EPT = E // NS
BS = 80
NCH_S = EPT // BS
RPT = 640
RPD = 80
NKD = RPT // RPD
NPAD = NS * RPT

BE = 4000
NBLK = E // BE


@functools.cache
def _build_gather(gbase, gsz):
    mesh = plsc.VectorSubcoreMesh(core_axis_name="c", subcore_axis_name="s")
    epw = gsz // NW
    nch = epw // BG
    assert epw % BG == 0 and epw % 8 == 0

    @functools.partial(
        pl.kernel,
        mesh=mesh,
        out_type=[jax.ShapeDtypeStruct((gsz, D), jnp.float32)] * 7,
        scratch_types=(
            [pltpu.VMEM((epw,), jnp.int32)] * 8
            + [pltpu.VMEM((BG, D), jnp.float32)] * 16
            + [pltpu.SemaphoreType.DMA] * 4
        ),
        compiler_params=pltpu.CompilerParams(needs_layout_passes=False),
    )
    def gather_kernel(user_hbm, item_hbm, qt_hbm,
                      src_i, dst_i, q0_i, q1_i, uq0_i, uq1_i, iq0_i, iq1_i,
                      o_src, o_dst, o_qs, o_uq0, o_uq1, o_iq0, o_iq1,
                      v_src, v_dst, v_q0, v_q1, v_uq0, v_uq1, v_iq0, v_iq1,
                      *bufs_and_sems):
        rows0 = bufs_and_sems[0:8]
        rows1 = bufs_and_sems[8:16]
        g0, g1, w0, w1 = bufs_and_sems[16:20]
        wid = lax.axis_index("s") * NC + lax.axis_index("c")
        base = wid * epw
        ivs = (v_src, v_dst, v_q0, v_q1, v_uq0, v_uq1, v_iq0, v_iq1)
        ihs = (src_i, dst_i, q0_i, q1_i, uq0_i, uq1_i, iq0_i, iq1_i)
        for iv, ih in zip(ivs, ihs):
            pltpu.async_copy(ih.at[pl.ds(gbase + base, epw)], iv, g1)
        for iv, ih in zip(ivs, ihs):
            pltpu.make_async_copy(ih.at[pl.ds(0, epw)], iv, g1).wait()

        tables = (user_hbm, item_hbm) + (qt_hbm,) * 6
        ohbm = (o_src, o_dst, o_qs, o_uq0, o_uq1, o_iq0, o_iq1)

        def fire_g(j, rows, sem):
            off = j * BG
            for tbl, iv, rv in zip(tables, ivs, rows):
                pltpu.async_copy(tbl.at[iv.at[pl.ds(off, BG)]], rv, sem)

        def wait_g(rows, sem):
            for rv in rows:
                pltpu.make_async_copy(qt_hbm.at[pl.ds(0, BG)], rv, sem).wait()

        def wbufs(rows):
            return (rows[0], rows[1], rows[2], rows[4], rows[5],
                    rows[6], rows[7])

        def fire_w(j, rows, sem):
            ob = base + j * BG
            for ov, rv in zip(ohbm, wbufs(rows)):
                pltpu.async_copy(rv, ov.at[pl.ds(ob, BG)], sem)

        def wait_w(rows, sem):
            for ov, rv in zip(ohbm, wbufs(rows)):
                pltpu.make_async_copy(rv, ov.at[pl.ds(0, BG)], sem).wait()

        def qsum(rows):
            rq0, rq1 = rows[2], rows[3]

            def row(r, c2):
                for c in range(D // LANES):
                    s = pl.ds(c * LANES, LANES)
                    rq0[r, s] = rq0[r, s] + rq1[r, s]
                return c2
            lax.fori_loop(0, BG, row, 0)

        fire_g(0, rows0, g0)
        nbody = (nch - 1) // 2 if nch % 2 == 1 else (nch - 2) // 2

        def body(i, carry):
            a = 2 * i
            b = a + 1

            @pl.when(i > 0)
            def _():
                wait_w(rows1, w1)
            fire_g(b, rows1, g1)
            wait_g(rows0, g0)
            qsum(rows0)
            fire_w(a, rows0, w0)
            wait_w(rows0, w0)
            fire_g(a + 2, rows0, g0)
            wait_g(rows1, g1)
            qsum(rows1)
            fire_w(b, rows1, w1)
            return carry
        lax.fori_loop(0, nbody, body, 0)

        if nch % 2 == 1:
            wait_w(rows1, w1)
            wait_g(rows0, g0)
            qsum(rows0)
            fire_w(nch - 1, rows0, w0)
            wait_w(rows0, w0)
        else:
            wait_w(rows1, w1)
            fire_g(nch - 1, rows1, g1)
            wait_g(rows0, g0)
            qsum(rows0)
            fire_w(nch - 2, rows0, w0)
            wait_g(rows1, g1)
            qsum(rows1)
            fire_w(nch - 1, rows1, w1)
            wait_w(rows0, w0)
            wait_w(rows1, w1)

    return gather_kernel


def _dense_body(se_ref, de_ref, qs_ref, u0_ref, u1_ref, i0_ref, i1_ref,
                mk_ref, wk_ref, bk_ref, w1_ref, b1_ref, w2_ref, b2_ref,
                o1_ref, o2_ref):
    src = se_ref[...]
    dst = de_ref[...]
    qs = qs_ref[...]
    u0 = u0_ref[...]
    u1 = u1_ref[...]
    i0 = i0_ref[...]
    i1 = i1_ref[...]
    mk = mk_ref[...]

    f32 = jnp.float32
    kkey = (jnp.dot(src, wk_ref[0:D, :], preferred_element_type=f32)
            + jnp.dot(u0, wk_ref[D:2 * D, :], preferred_element_type=f32)
            + jnp.dot(u1, wk_ref[2 * D:3 * D, :], preferred_element_type=f32)
            + bk_ref[...])
    inv_sqrt_d = 1.0 / (D ** 0.5)
    s0 = jnp.sum(i0 * kkey, axis=1, keepdims=True) * inv_sqrt_d
    s1 = jnp.sum(i1 * kkey, axis=1, keepdims=True) * inv_sqrt_d
    m = jnp.maximum(s0, s1)
    e0 = jnp.exp(s0 - m)
    e1 = jnp.exp(s1 - m)
    den = e0 + e1
    a0 = e0 / den
    a1 = e1 / den
    iqe = a0 * i0 + a1 * i1
    uqs = u0 + u1

    h = (jnp.dot(src, w1_ref[0:D, :], preferred_element_type=f32)
         + jnp.dot(dst, w1_ref[D:2 * D, :], preferred_element_type=f32)
         + jnp.dot(uqs, w1_ref[2 * D:3 * D, :], preferred_element_type=f32)
         + jnp.dot(iqe, w1_ref[3 * D:4 * D, :], preferred_element_type=f32)
         + b1_ref[...])
    h = jnp.maximum(h, 0.0)
    rec = jnp.dot(h, w2_ref[...], preferred_element_type=f32) + b2_ref[...]
    qe = jnp.where(mk > 0.5, rec, qs)
    o1_ref[...] = src + qe
    o2_ref[...] = dst - qe


@functools.cache
def _build_dense(esz):
    return pl.pallas_call(
        _dense_body,
        grid=(esz // BE,),
        in_specs=[pl.BlockSpec((BE, D), lambda i: (i, 0))] * 7
        + [pl.BlockSpec((BE, 1), lambda i: (i, 0)),
           pl.BlockSpec((3 * D, D), lambda i: (0, 0)),
           pl.BlockSpec((1, D), lambda i: (0, 0)),
           pl.BlockSpec((4 * D, H), lambda i: (0, 0)),
           pl.BlockSpec((1, H), lambda i: (0, 0)),
           pl.BlockSpec((H, D), lambda i: (0, 0)),
           pl.BlockSpec((1, D), lambda i: (0, 0))],
        out_specs=[pl.BlockSpec((BE, D), lambda i: (i, 0))] * 2,
        out_shape=[jax.ShapeDtypeStruct((esz, D), jnp.float32)] * 2,
    )


@functools.cache
def _build_scatter():
    mesh = plsc.VectorSubcoreMesh(core_axis_name="c", subcore_axis_name="s")

    @functools.partial(
        pl.kernel,
        mesh=mesh,
        out_type=[jax.ShapeDtypeStruct((N_NODES, D), jnp.float32),
                  jax.ShapeDtypeStruct((N_NODES, D), jnp.float32)],
        scratch_types=[
            pltpu.VMEM((1, BS), jnp.int32),
            pltpu.VMEM((1, BS), jnp.int32),
            pltpu.VMEM((BS, D), jnp.float32),
            pltpu.VMEM((BS, D), jnp.float32),
            pltpu.VMEM((NPAD,), jnp.float32),
            pltpu.VMEM((RPT,), jnp.float32),
            pltpu.VMEM((RPT,), jnp.float32),
            pltpu.VMEM_SHARED((N_NODES, D), jnp.float32),
        ] + [pltpu.VMEM_SHARED((NPAD,), jnp.float32)] * NS
          + [pltpu.SemaphoreType.DMA] * 4,
        compiler_params=pltpu.CompilerParams(needs_layout_passes=False),
    )
    def scatter_kernel(*refs):
        mu = refs[0:NSPL]
        mi = refs[NSPL:2 * NSPL]
        dst_f, src_f = refs[2 * NSPL:2 * NSPL + 2]
        (user_out, item_out, idx0_v, idx1_v, msg0_v, msg1_v, cnt_v,
         tmp_v, ctot_v, acc_sh) = refs[2 * NSPL + 2:2 * NSPL + 12]
        stage_and_sems = refs[2 * NSPL + 12:]
        out_v = msg0_v
        stage = stage_and_sems[:NS]
        l0, l1, s0, s1 = stage_and_sems[NS:NS + 4]
        core = lax.axis_index("c")
        tid = lax.axis_index("s")
        zero16 = jnp.zeros((LANES,), jnp.float32)
        one16 = jnp.ones((LANES,), jnp.float32)

        def zrow(r, c2):
            for c in range(D // LANES):
                out_v[r, pl.ds(c * LANES, LANES)] = zero16
            return c2
        lax.fori_loop(0, RPD, zrow, 0)

        def zcnt(r, c2):
            cnt_v[pl.ds(r * LANES, LANES)] = zero16
            return c2
        lax.fori_loop(0, NPAD // LANES, zcnt, 0)

        rbase = tid * RPT
        for k in range(NKD):
            @pl.when(rbase + k * RPD < N_NODES)
            def _():
                pltpu.sync_copy(out_v, acc_sh.at[pl.ds(rbase + k * RPD, RPD)])
        plsc.subcore_barrier()

        def side(m_parts, idx_hbm, out_hbm):
            ebase = tid * EPT

            def fire_l(j, iv, mv, sem):
                off = ebase + j * BS
                pltpu.async_copy(idx_hbm.at[pl.ds(off, BS)], iv.at[0], sem)
                for s in range(NSPL):
                    @pl.when((off >= SPLITS[s]) & (off < SPLITS[s + 1]))
                    def _():
                        pltpu.async_copy(
                            m_parts[s].at[pl.ds(off - SPLITS[s], BS)], mv,
                            sem)

            def wait_l(iv, mv, sem):
                pltpu.make_async_copy(idx_hbm.at[pl.ds(0, BS)], iv.at[0],
                                      sem).wait()
                pltpu.make_async_copy(m_parts[0].at[pl.ds(0, BS)], mv,
                                      sem).wait()

            def counts(iv):
                for v in range(BS // LANES):
                    vec = iv[0, pl.ds(v * LANES, LANES)]
                    plsc.addupdate_scatter(cnt_v, [vec], one16)

            def fire_s(iv, mv, sem):
                pltpu.async_copy(mv, acc_sh.at[iv.at[0]], sem, add=True)

            def wait_s(mv, sem):
                pltpu.make_async_copy(m_parts[0].at[pl.ds(0, BS)], mv,
                                      sem).wait()

            fire_l(0, idx0_v, msg0_v, l0)

            def chunk2(i, c2):
                a = 2 * i

                @pl.when(i > 0)
                def _():
                    wait_s(msg1_v, s1)
                fire_l(a + 1, idx1_v, msg1_v, l1)
                wait_l(idx0_v, msg0_v, l0)
                counts(idx0_v)
                fire_s(idx0_v, msg0_v, s0)
                wait_s(msg0_v, s0)
                fire_l(a + 2, idx0_v, msg0_v, l0)
                wait_l(idx1_v, msg1_v, l1)
                counts(idx1_v)
                fire_s(idx1_v, msg1_v, s1)
                return c2
            lax.fori_loop(0, (NCH_S - 1) // 2, chunk2, 0)

            wait_s(msg1_v, s1)
            wait_l(idx0_v, msg0_v, l0)
            counts(idx0_v)
            fire_s(idx0_v, msg0_v, s0)
            wait_s(msg0_v, s0)

            for i in range(NS):
                @pl.when(tid == i)
                def _():
                    pltpu.sync_copy(cnt_v, stage[i])
            plsc.subcore_barrier()

            def zc(r, c2):
                ctot_v[pl.ds(r * LANES, LANES)] = zero16
                return c2
            lax.fori_loop(0, RPT // LANES, zc, 0)
            for i in range(NS):
                pltpu.sync_copy(stage[i].at[pl.ds(rbase, RPT)], tmp_v)

                def acc_cnt(r, c2):
                    s = pl.ds(r * LANES, LANES)
                    ctot_v[s] = ctot_v[s] + tmp_v[s]
                    return c2
                lax.fori_loop(0, RPT // LANES, acc_cnt, 0)

            def rowdiv_at(k):
                def rowdiv(r, c2):
                    iv = jnp.zeros((LANES,), jnp.int32) + (k * RPD + r)
                    c = plsc.load_gather(ctot_v, [iv])
                    invc = 1.0 / jnp.maximum(c, 1.0)
                    for cc in range(D // LANES):
                        s = pl.ds(cc * LANES, LANES)
                        out_v[r, s] = out_v[r, s] * invc
                    return c2
                return rowdiv

            for k in range(NKD):
                rb = rbase + k * RPD

                @pl.when(rb < N_NODES)
                def _():
                    pltpu.sync_copy(acc_sh.at[pl.ds(rb, RPD)], out_v)
                    lax.fori_loop(0, RPD, rowdiv_at(k), 0)
                    pltpu.sync_copy(out_v, out_hbm.at[pl.ds(rb, RPD)])

        @pl.when(core == 0)
        def _():
            side(mu, dst_f, item_out)

        @pl.when(core == 1)
        def _():
            side(mi, src_f, user_out)

    return scatter_kernel


def kernel(user_emb, item_emb, query_table, W_K, b_K, W1, b1, W2, b2,
           edge_index, edge_type, query_idx, userquery_idx, itemquery_idx):
    i32 = jnp.int32
    src = edge_index[0].astype(i32)
    dst = edge_index[1].astype(i32)
    q0 = query_idx[:, 0].astype(i32)
    q1 = query_idx[:, 1].astype(i32)
    uq0 = userquery_idx[:, 0].astype(i32)
    uq1 = userquery_idx[:, 1].astype(i32)
    iq0 = itemquery_idx[:, 0].astype(i32)
    iq1 = itemquery_idx[:, 1].astype(i32)

    mask = (edge_type == 0).astype(jnp.float32)[:, None]
    wargs = (W_K, b_K.reshape(1, D), W1, b1.reshape(1, H), W2,
             b2.reshape(1, D))

    mus, mis = [], []
    for s in range(NSPL):
        gbase, gsz = SPLITS[s], SPLITS[s + 1] - SPLITS[s]
        parts = _build_gather(gbase, gsz)(
            user_emb, item_emb, query_table,
            src, dst, q0, q1, uq0, uq1, iq0, iq1)
        m_u, m_i = _build_dense(gsz)(
            *parts, lax.slice_in_dim(mask, gbase, gbase + gsz), *wargs)
        mus.append(m_u)
        mis.append(m_i)

    user_out, item_out = _build_scatter()(*mus, *mis, dst, src)
    return user_out, item_out
